# baseline (speedup 1.0000x reference)
import dataclasses
import functools

import jax
import jax.numpy as jnp
from jax import lax
from jax.experimental import pallas as pl
from jax.experimental.pallas import tpu as pltpu
from jax.experimental.pallas import tpu_sc as plsc

NC = 2
NS = 16
NW = NC * NS
B = 128

N = 10000
E = 320000
K = 80
K2 = 2 * K
E_PAD = NW * K * B
N_PAD = 10112
RPT = N_PAD // NS


def _mesh():
    return plsc.VectorSubcoreMesh(core_axis_name="c", subcore_axis_name="s")


def _sc_params():
    cp = pltpu.CompilerParams()
    fields = pltpu.CompilerParams.__dataclass_fields__
    if "needs_layout_passes" in fields:
        cp = dataclasses.replace(cp, needs_layout_passes=False)
    if "use_tc_tiling_on_sc" in fields:
        cp = dataclasses.replace(cp, use_tc_tiling_on_sc=False)
    return cp


def _deg_pass(dst3d, zeros_col):
    nch = N_PAD // B

    @functools.partial(
        pl.kernel,
        out_type=jax.ShapeDtypeStruct((NC, N, 1), jnp.float32),
        mesh=_mesh(),
        scratch_types=[
            pltpu.VMEM((K, B), jnp.int32),
            pltpu.VMEM((N_PAD, 1), jnp.float32),
            pltpu.VMEM((B,), jnp.int32),
            pltpu.VMEM_SHARED((N_PAD, 1), jnp.float32),
        ],
        compiler_params=_sc_params(),
    )
    def k(dst_hbm, zeros_hbm, out_hbm, dst_v, hist, ids, acc):
        c = lax.axis_index("c")
        s = lax.axis_index("s")
        wid = c * NS + s
        pltpu.sync_copy(dst_hbm.at[wid], dst_v)
        pltpu.sync_copy(zeros_hbm, hist)
        pltpu.sync_copy(zeros_hbm.at[pl.ds(s * RPT, RPT)],
                        acc.at[pl.ds(s * RPT, RPT)])
        plsc.subcore_barrier()
        ones16 = jnp.ones((16,), jnp.float32)
        zero16 = jnp.zeros((16,), jnp.int32)

        def body(j, carry):
            for l in range(B // 16):
                idx = dst_v[j, pl.ds(l * 16, 16)]
                plsc.addupdate_scatter(hist, [idx, zero16], ones16)
            return carry

        lax.fori_loop(0, K, body, 0)

        def reduce(m, carry):
            ch = s + NS * m

            @pl.when(ch < nch)
            def _():
                for l in range(B // 16):
                    ids[pl.ds(l * 16, 16)] = (ch * B + l * 16
                                              + lax.iota(jnp.int32, 16))
                pltpu.sync_copy(hist.at[pl.ds(ch * B, B)], acc.at[ids],
                                add=True)

            return carry

        lax.fori_loop(0, (nch + NS - 1) // NS, reduce, 0)
        plsc.subcore_barrier()
        pltpu.sync_copy(acc.at[pl.ds(s * NTR, NTR)],
                        out_hbm.at[c, pl.ds(s * NTR, NTR)])

    return k(dst3d, zeros_col)


W = 4
SB = 16
NTR = N // NS


def _sb_pipeline(tbl, acc, idx, rows, semg, sems):
    for b in range(W):
        pltpu.async_copy(tbl.at[idx.at[b, 0]], rows.at[b], semg.at[b])
    rounds = SB // W
    for r in range(rounds):
        for b in range(W):
            t = r * W + b
            pltpu.make_async_copy(tbl.at[idx.at[t, 0]], rows.at[b],
                                  semg.at[b]).wait()
            pltpu.async_copy(rows.at[b], acc.at[idx.at[t, 1]], sems.at[b],
                             add=True)
        for b in range(W):
            t = r * W + b
            if t + W < SB:
                pltpu.make_async_copy(rows.at[b], acc.at[idx.at[t, 1]],
                                      sems.at[b]).wait()
                pltpu.async_copy(tbl.at[idx.at[t + W, 0]], rows.at[b],
                                 semg.at[b])
    for b in range(W):
        t = (rounds - 1) * W + b
        pltpu.make_async_copy(rows.at[b], acc.at[idx.at[t, 1]],
                              sems.at[b]).wait()


def _seg_pass(table, cmb, zero_rows, d, nsb, feat):
    @functools.partial(
        pl.kernel,
        out_type=jax.ShapeDtypeStruct((NC, N, d), jnp.float32),
        mesh=_mesh(),
        scratch_types=[
            pltpu.VMEM((2, SB, 2, B), jnp.int32),
            pltpu.VMEM((W, B, d), jnp.float32),
            pltpu.VMEM_SHARED((N, d), jnp.float32),
            pltpu.VMEM_SHARED((N_PAD, d), jnp.float32),
            pltpu.SemaphoreType.DMA,
            pltpu.SemaphoreType.DMA((W,)),
            pltpu.SemaphoreType.DMA((W,)),
        ],
        compiler_params=_sc_params(),
    )
    def k(tbl_hbm, cmb_hbm, zeros_hbm, out_hbm,
          idxb, rows, tbl, acc, semi, semg, sems):
        c = lax.axis_index("c")
        s = lax.axis_index("s")
        grp = s if feat else c * NS + s
        if feat:
            stage_src = tbl_hbm.at[c, pl.ds(s * NTR, NTR)]
        else:
            stage_src = tbl_hbm.at[pl.ds(s * NTR, NTR)]
        pltpu.sync_copy(stage_src, tbl.at[pl.ds(s * NTR, NTR)])
        pltpu.sync_copy(zeros_hbm, acc.at[pl.ds(s * RPT, RPT)])
        pltpu.sync_copy(cmb_hbm.at[grp, pl.ds(0, SB)], idxb.at[0])
        plsc.subcore_barrier()

        def body(q, carry):
            pq = jnp.bitwise_and(q, 1)

            @pl.when(q < nsb - 1)
            def _start():
                pltpu.async_copy(cmb_hbm.at[grp, pl.ds((q + 1) * SB, SB)],
                                 idxb.at[1 - pq], semi)

            _sb_pipeline(tbl, acc, idxb.at[pq], rows, semg, sems)

            @pl.when(q < nsb - 1)
            def _wait():
                pltpu.make_async_copy(cmb_hbm.at[grp, pl.ds((q + 1) * SB, SB)],
                                      idxb.at[1 - pq], semi).wait()

            return carry

        lax.fori_loop(0, nsb, body, 0)
        plsc.subcore_barrier()
        pltpu.sync_copy(acc.at[pl.ds(s * NTR, NTR)],
                        out_hbm.at[c, pl.ds(s * NTR, NTR)])

    return k(table, cmb, zero_rows)


_ROWS = 2000


def _k0(x, w1):
    def body(x_ref, w_ref, o_ref):
        o_ref[...] = jax.lax.dot_general(
            x_ref[...], w_ref[...], (((1,), (0,)), ((), ())),
            preferred_element_type=jnp.float32,
            precision=jax.lax.Precision.HIGHEST)

    d_in, d_h = w1.shape
    return pl.pallas_call(
        body,
        grid=(N // _ROWS,),
        in_specs=[
            pl.BlockSpec((_ROWS, d_in), lambda i: (i, 0)),
            pl.BlockSpec((d_in, d_h), lambda i: (0, 0)),
        ],
        out_specs=pl.BlockSpec((_ROWS, d_h), lambda i: (i, 0)),
        out_shape=jax.ShapeDtypeStruct((N, d_h), jnp.float32),
    )(x, w1)


def _k1(p1, deg_cols):
    def body(p_ref, deg_ref, o_ref, dinv_ref):
        dinv = lax.rsqrt(deg_ref[0] + deg_ref[1] + 1.0)
        dinv_ref[...] = dinv
        g = p_ref[...] * dinv
        h = p_ref.shape[1] // 2
        o_ref[0] = g[:, :h]
        o_ref[1] = g[:, h:]

    d_h = p1.shape[1]
    return pl.pallas_call(
        body,
        grid=(N // _ROWS,),
        in_specs=[
            pl.BlockSpec((_ROWS, d_h), lambda i: (i, 0)),
            pl.BlockSpec((2, _ROWS, 1), lambda i: (0, i, 0)),
        ],
        out_specs=[
            pl.BlockSpec((2, _ROWS, d_h // 2), lambda i: (0, i, 0)),
            pl.BlockSpec((_ROWS, 1), lambda i: (i, 0)),
        ],
        out_shape=[
            jax.ShapeDtypeStruct((2, N, d_h // 2), jnp.float32),
            jax.ShapeDtypeStruct((N, 1), jnp.float32),
        ],
    )(p1, deg_cols)


def _k2(s1, g1s, dinv, b1, w2):
    def body(s_ref, g_ref, dinv_ref, b_ref, w_ref, o_ref):
        dv = dinv_ref[...]
        full = jnp.concatenate([s_ref[0] + g_ref[0], s_ref[1] + g_ref[1]],
                               axis=-1)
        h = dv * full + b_ref[...]
        h = jnp.maximum(h, 0.0)
        p = jax.lax.dot_general(h, w_ref[...], (((1,), (0,)), ((), ())),
                                preferred_element_type=jnp.float32,
                                precision=jax.lax.Precision.HIGHEST)
        o_ref[...] = p * dv

    d_h, d_o = w2.shape
    return pl.pallas_call(
        body,
        grid=(N // _ROWS,),
        in_specs=[
            pl.BlockSpec((2, _ROWS, d_h // 2), lambda i: (0, i, 0)),
            pl.BlockSpec((2, _ROWS, d_h // 2), lambda i: (0, i, 0)),
            pl.BlockSpec((_ROWS, 1), lambda i: (i, 0)),
            pl.BlockSpec((1, d_h), lambda i: (0, 0)),
            pl.BlockSpec((d_h, d_o), lambda i: (0, 0)),
        ],
        out_specs=pl.BlockSpec((_ROWS, d_o), lambda i: (i, 0)),
        out_shape=jax.ShapeDtypeStruct((N, d_o), jnp.float32),
    )(s1, g1s, dinv, b1, w2)


def _k3(s2, g2, dinv, b2):
    def body(s_ref, g_ref, dinv_ref, b_ref, o_ref):
        o_ref[...] = (dinv_ref[...] * (s_ref[0] + s_ref[1] + g_ref[...])
                      + b_ref[...])

    d_o = g2.shape[1]
    return pl.pallas_call(
        body,
        grid=(N // _ROWS,),
        in_specs=[
            pl.BlockSpec((2, _ROWS, d_o), lambda i: (0, i, 0)),
            pl.BlockSpec((_ROWS, d_o), lambda i: (i, 0)),
            pl.BlockSpec((_ROWS, 1), lambda i: (i, 0)),
            pl.BlockSpec((1, d_o), lambda i: (0, 0)),
        ],
        out_specs=pl.BlockSpec((_ROWS, d_o), lambda i: (i, 0)),
        out_shape=jax.ShapeDtypeStruct((N, d_o), jnp.float32),
    )(s2, g2, dinv, b2)


def kernel(x, edge_index, W1, b1, W2, b2):
    d_h = W1.shape[1]
    d_o = W2.shape[1]
    pad = E_PAD - E
    src = jnp.concatenate([edge_index[0], jnp.zeros((pad,), jnp.int32)])
    dst = jnp.concatenate([edge_index[1], jnp.full((pad,), N, jnp.int32)])
    dst3d = dst.reshape(NW, K, B)
    cmb = jnp.stack([src.reshape(NS, K2, B), dst.reshape(NS, K2, B)],
                    axis=2)
    cmb_edge = cmb.reshape(NW, K, 2, B)

    zeros_h = jnp.zeros((RPT, d_h // 2), jnp.float32)
    zeros_o = jnp.zeros((RPT, d_o), jnp.float32)
    zeros_col = jnp.zeros((N_PAD, 1), jnp.float32)

    deg_cols = _deg_pass(dst3d, zeros_col)

    p1 = _k0(x, W1)
    g1s, dinv = _k1(p1, deg_cols)
    s1 = _seg_pass(g1s, cmb, zeros_h, d_h // 2, K2 // SB, True)
    g2 = _k2(s1, g1s, dinv, b1.reshape(1, d_h), W2)
    s2 = _seg_pass(g2, cmb_edge, zeros_o, d_o, K // SB, False)
    return _k3(s2, g2, dinv, b2.reshape(1, d_o))


# speedup vs baseline: 1.0447x; 1.0447x over previous
import dataclasses
import functools

import jax
import jax.numpy as jnp
from jax import lax
from jax.experimental import pallas as pl
from jax.experimental.pallas import tpu as pltpu
from jax.experimental.pallas import tpu_sc as plsc

NC = 2
NS = 16
NW = NC * NS
B = 128

N = 10000
E = 320000
K = 80
K2 = 2 * K
E_PAD = NW * K * B
N_PAD = 10112
RPT = N_PAD // NS


def _mesh():
    return plsc.VectorSubcoreMesh(core_axis_name="c", subcore_axis_name="s")


def _sc_params():
    cp = pltpu.CompilerParams()
    fields = pltpu.CompilerParams.__dataclass_fields__
    if "needs_layout_passes" in fields:
        cp = dataclasses.replace(cp, needs_layout_passes=False)
    if "use_tc_tiling_on_sc" in fields:
        cp = dataclasses.replace(cp, use_tc_tiling_on_sc=False)
    return cp


def _deg_pass(dst3d):
    @functools.partial(
        pl.kernel,
        out_type=jax.ShapeDtypeStruct((NW, N), jnp.float32),
        mesh=_mesh(),
        scratch_types=[
            pltpu.VMEM((K, B), jnp.int32),
            pltpu.VMEM((N_PAD,), jnp.float32),
        ],
        compiler_params=_sc_params(),
    )
    def k(dst_hbm, out_hbm, dst_v, hist):
        c = lax.axis_index("c")
        s = lax.axis_index("s")
        wid = c * NS + s
        pltpu.sync_copy(dst_hbm.at[wid], dst_v)

        def zero(i, carry):
            hist[pl.ds(i * 16, 16)] = jnp.zeros((16,), jnp.float32)
            return carry

        lax.fori_loop(0, N_PAD // 16, zero, 0)
        ones16 = jnp.ones((16,), jnp.float32)

        def body(j, carry):
            for l in range(B // 16):
                idx = dst_v[j, pl.ds(l * 16, 16)]
                plsc.addupdate_scatter(hist, [idx], ones16)
            return carry

        lax.fori_loop(0, K, body, 0)
        pltpu.sync_copy(hist.at[pl.ds(0, N)], out_hbm.at[wid])

    return k(dst3d)


W = 4
SB = 16
NTR = N // NS


def _sb_pipeline(tbl, acc, idx, idx_next, rows, semg, sems, has_next, on_mid):
    rounds = SB // W
    for r in range(rounds):
        for b in range(W):
            t = r * W + b
            pltpu.make_async_copy(tbl.at[idx.at[t, 0]], rows.at[b],
                                  semg.at[b]).wait()
            pltpu.async_copy(rows.at[b], acc.at[idx.at[t, 1]], sems.at[b],
                             add=True)
        if r == rounds - 1:
            on_mid()
        for b in range(W):
            t = r * W + b
            pltpu.make_async_copy(rows.at[b], acc.at[idx.at[t, 1]],
                                  sems.at[b]).wait()
            if t + W < SB:
                pltpu.async_copy(tbl.at[idx.at[t + W, 0]], rows.at[b],
                                 semg.at[b])
            else:

                @pl.when(has_next)
                def _(b=b, t=t):
                    pltpu.async_copy(tbl.at[idx_next.at[t + W - SB, 0]],
                                     rows.at[b], semg.at[b])


def _seg_pass(table, cmb, zero_rows, d, nsb, feat):
    @functools.partial(
        pl.kernel,
        out_type=jax.ShapeDtypeStruct((NC, N, d), jnp.float32),
        mesh=_mesh(),
        scratch_types=[
            pltpu.VMEM((2, SB, 2, B), jnp.int32),
            pltpu.VMEM((W, B, d), jnp.float32),
            pltpu.VMEM_SHARED((N, d), jnp.float32),
            pltpu.VMEM_SHARED((N_PAD, d), jnp.float32),
            pltpu.SemaphoreType.DMA,
            pltpu.SemaphoreType.DMA((W,)),
            pltpu.SemaphoreType.DMA((W,)),
        ],
        compiler_params=_sc_params(),
    )
    def k(tbl_hbm, cmb_hbm, zeros_hbm, out_hbm,
          idxb, rows, tbl, acc, semi, semg, sems):
        c = lax.axis_index("c")
        s = lax.axis_index("s")
        grp = s if feat else c * NS + s
        if feat:
            stage_src = tbl_hbm.at[c, pl.ds(s * NTR, NTR)]
        else:
            stage_src = tbl_hbm.at[pl.ds(s * NTR, NTR)]
        pltpu.sync_copy(stage_src, tbl.at[pl.ds(s * NTR, NTR)])
        pltpu.sync_copy(zeros_hbm, acc.at[pl.ds(s * RPT, RPT)])
        pltpu.sync_copy(cmb_hbm.at[grp, pl.ds(0, SB)], idxb.at[0])
        plsc.subcore_barrier()
        for b in range(W):
            pltpu.async_copy(tbl.at[idxb.at[0, b, 0]], rows.at[b], semg.at[b])

        def body(q, carry):
            pq = jnp.bitwise_and(q, 1)
            has_next = q < nsb - 1

            @pl.when(has_next)
            def _start():
                pltpu.async_copy(cmb_hbm.at[grp, pl.ds((q + 1) * SB, SB)],
                                 idxb.at[1 - pq], semi)

            def on_mid():
                @pl.when(has_next)
                def _wait():
                    pltpu.make_async_copy(
                        cmb_hbm.at[grp, pl.ds((q + 1) * SB, SB)],
                        idxb.at[1 - pq], semi).wait()

            _sb_pipeline(tbl, acc, idxb.at[pq], idxb.at[1 - pq], rows,
                         semg, sems, has_next, on_mid)
            return carry

        lax.fori_loop(0, nsb, body, 0)
        plsc.subcore_barrier()
        pltpu.sync_copy(acc.at[pl.ds(s * NTR, NTR)],
                        out_hbm.at[c, pl.ds(s * NTR, NTR)])

    return k(table, cmb, zero_rows)


_ROWS = 2000


def _k0(x, w1):
    def body(x_ref, w_ref, o_ref):
        o_ref[...] = jax.lax.dot_general(
            x_ref[...], w_ref[...], (((1,), (0,)), ((), ())),
            preferred_element_type=jnp.float32,
            precision=jax.lax.Precision.HIGHEST)

    d_in, d_h = w1.shape
    return pl.pallas_call(
        body,
        grid=(N // _ROWS,),
        in_specs=[
            pl.BlockSpec((_ROWS, d_in), lambda i: (i, 0)),
            pl.BlockSpec((d_in, d_h), lambda i: (0, 0)),
        ],
        out_specs=pl.BlockSpec((_ROWS, d_h), lambda i: (i, 0)),
        out_shape=jax.ShapeDtypeStruct((N, d_h), jnp.float32),
    )(x, w1)


def _k1(p1, degt):
    def body(p_ref, deg_ref, o_ref, dinv_ref):
        dinv = lax.rsqrt(jnp.sum(deg_ref[...], axis=1, keepdims=True) + 1.0)
        dinv_ref[...] = dinv
        g = p_ref[...] * dinv
        h = p_ref.shape[1] // 2
        o_ref[0] = g[:, :h]
        o_ref[1] = g[:, h:]

    d_h = p1.shape[1]
    return pl.pallas_call(
        body,
        grid=(N // _ROWS,),
        in_specs=[
            pl.BlockSpec((_ROWS, d_h), lambda i: (i, 0)),
            pl.BlockSpec((_ROWS, NW), lambda i: (i, 0)),
        ],
        out_specs=[
            pl.BlockSpec((2, _ROWS, d_h // 2), lambda i: (0, i, 0)),
            pl.BlockSpec((_ROWS, 1), lambda i: (i, 0)),
        ],
        out_shape=[
            jax.ShapeDtypeStruct((2, N, d_h // 2), jnp.float32),
            jax.ShapeDtypeStruct((N, 1), jnp.float32),
        ],
    )(p1, degt)


def _k2(s1, g1s, dinv, b1, w2):
    def body(s_ref, g_ref, dinv_ref, b_ref, w_ref, o_ref):
        dv = dinv_ref[...]
        full = jnp.concatenate([s_ref[0] + g_ref[0], s_ref[1] + g_ref[1]],
                               axis=-1)
        h = dv * full + b_ref[...]
        h = jnp.maximum(h, 0.0)
        p = jax.lax.dot_general(h, w_ref[...], (((1,), (0,)), ((), ())),
                                preferred_element_type=jnp.float32,
                                precision=jax.lax.Precision.HIGHEST)
        o_ref[...] = p * dv

    d_h, d_o = w2.shape
    return pl.pallas_call(
        body,
        grid=(N // _ROWS,),
        in_specs=[
            pl.BlockSpec((2, _ROWS, d_h // 2), lambda i: (0, i, 0)),
            pl.BlockSpec((2, _ROWS, d_h // 2), lambda i: (0, i, 0)),
            pl.BlockSpec((_ROWS, 1), lambda i: (i, 0)),
            pl.BlockSpec((1, d_h), lambda i: (0, 0)),
            pl.BlockSpec((d_h, d_o), lambda i: (0, 0)),
        ],
        out_specs=pl.BlockSpec((_ROWS, d_o), lambda i: (i, 0)),
        out_shape=jax.ShapeDtypeStruct((N, d_o), jnp.float32),
    )(s1, g1s, dinv, b1, w2)


def _k3(s2, g2, dinv, b2):
    def body(s_ref, g_ref, dinv_ref, b_ref, o_ref):
        o_ref[...] = (dinv_ref[...] * (s_ref[0] + s_ref[1] + g_ref[...])
                      + b_ref[...])

    d_o = g2.shape[1]
    return pl.pallas_call(
        body,
        grid=(N // _ROWS,),
        in_specs=[
            pl.BlockSpec((2, _ROWS, d_o), lambda i: (0, i, 0)),
            pl.BlockSpec((_ROWS, d_o), lambda i: (i, 0)),
            pl.BlockSpec((_ROWS, 1), lambda i: (i, 0)),
            pl.BlockSpec((1, d_o), lambda i: (0, 0)),
        ],
        out_specs=pl.BlockSpec((_ROWS, d_o), lambda i: (i, 0)),
        out_shape=jax.ShapeDtypeStruct((N, d_o), jnp.float32),
    )(s2, g2, dinv, b2)


def kernel(x, edge_index, W1, b1, W2, b2):
    d_h = W1.shape[1]
    d_o = W2.shape[1]
    pad = E_PAD - E
    src = jnp.concatenate([edge_index[0], jnp.zeros((pad,), jnp.int32)])
    dst = jnp.concatenate([edge_index[1], jnp.full((pad,), N, jnp.int32)])
    dst3d = dst.reshape(NW, K, B)
    cmb = jnp.stack([src.reshape(NS, K2, B), dst.reshape(NS, K2, B)],
                    axis=2)
    cmb_edge = cmb.reshape(NW, K, 2, B)

    zeros_h = jnp.zeros((RPT, d_h // 2), jnp.float32)
    zeros_o = jnp.zeros((RPT, d_o), jnp.float32)

    deg_parts = _deg_pass(dst3d)
    degt = deg_parts.T

    p1 = _k0(x, W1)
    g1s, dinv = _k1(p1, degt)
    s1 = _seg_pass(g1s, cmb, zeros_h, d_h // 2, K2 // SB, True)
    g2 = _k2(s1, g1s, dinv, b1.reshape(1, d_h), W2)
    s2 = _seg_pass(g2, cmb_edge, zeros_o, d_o, K // SB, False)
    return _k3(s2, g2, dinv, b2.reshape(1, d_o))


# speedup vs baseline: 1.0628x; 1.0173x over previous
import dataclasses
import functools

import jax
import jax.numpy as jnp
from jax import lax
from jax.experimental import pallas as pl
from jax.experimental.pallas import tpu as pltpu
from jax.experimental.pallas import tpu_sc as plsc

NC = 2
NS = 16
NW = NC * NS
B = 128

N = 10000
E = 320000
K = 80
K2 = 2 * K
E_PAD = NW * K * B
N_PAD = 10112
RPT = N_PAD // NS


def _mesh():
    return plsc.VectorSubcoreMesh(core_axis_name="c", subcore_axis_name="s")


def _sc_params():
    cp = pltpu.CompilerParams()
    fields = pltpu.CompilerParams.__dataclass_fields__
    if "needs_layout_passes" in fields:
        cp = dataclasses.replace(cp, needs_layout_passes=False)
    if "use_tc_tiling_on_sc" in fields:
        cp = dataclasses.replace(cp, use_tc_tiling_on_sc=False)
    return cp


def _deg_pass(dst3d):
    @functools.partial(
        pl.kernel,
        out_type=jax.ShapeDtypeStruct((NW, N), jnp.float32),
        mesh=_mesh(),
        scratch_types=[
            pltpu.VMEM((K, B), jnp.int32),
            pltpu.VMEM((N_PAD,), jnp.float32),
        ],
        compiler_params=_sc_params(),
    )
    def k(dst_hbm, out_hbm, dst_v, hist):
        c = lax.axis_index("c")
        s = lax.axis_index("s")
        wid = c * NS + s
        pltpu.sync_copy(dst_hbm.at[wid], dst_v)

        def zero(i, carry):
            hist[pl.ds(i * 16, 16)] = jnp.zeros((16,), jnp.float32)
            return carry

        lax.fori_loop(0, N_PAD // 16, zero, 0)
        ones16 = jnp.ones((16,), jnp.float32)

        def body(j, carry):
            for l in range(B // 16):
                idx = dst_v[j, pl.ds(l * 16, 16)]
                plsc.addupdate_scatter(hist, [idx], ones16)
            return carry

        lax.fori_loop(0, K, body, 0)
        pltpu.sync_copy(hist.at[pl.ds(0, N)], out_hbm.at[wid])

    return k(dst3d)


W = 4
SB = 16
NTR = N // NS


def _sb_pipeline(tbl, acc, idx, rows, semg, sems):
    for b in range(W):
        pltpu.async_copy(tbl.at[idx.at[b, 0]], rows.at[b], semg.at[b])
    rounds = SB // W
    for r in range(rounds):
        for b in range(W):
            t = r * W + b
            pltpu.make_async_copy(tbl.at[idx.at[t, 0]], rows.at[b],
                                  semg.at[b]).wait()
            pltpu.async_copy(rows.at[b], acc.at[idx.at[t, 1]], sems.at[b],
                             add=True)
        for b in range(W):
            t = r * W + b
            if t + W < SB:
                pltpu.make_async_copy(rows.at[b], acc.at[idx.at[t, 1]],
                                      sems.at[b]).wait()
                pltpu.async_copy(tbl.at[idx.at[t + W, 0]], rows.at[b],
                                 semg.at[b])
    for b in range(W):
        t = (rounds - 1) * W + b
        pltpu.make_async_copy(rows.at[b], acc.at[idx.at[t, 1]],
                              sems.at[b]).wait()


def _seg_pass(table, cmb, zero_rows, d, nsb, feat):
    @functools.partial(
        pl.kernel,
        out_type=jax.ShapeDtypeStruct((NC, N, d), jnp.float32),
        mesh=_mesh(),
        scratch_types=[
            pltpu.VMEM((2, SB, 2, B), jnp.int32),
            pltpu.VMEM((W, B, d), jnp.float32),
            pltpu.VMEM_SHARED((N, d), jnp.float32),
            pltpu.VMEM_SHARED((N_PAD, d), jnp.float32),
            pltpu.SemaphoreType.DMA,
            pltpu.SemaphoreType.DMA((W,)),
            pltpu.SemaphoreType.DMA((W,)),
        ],
        compiler_params=_sc_params(),
    )
    def k(tbl_hbm, cmb_hbm, zeros_hbm, out_hbm,
          idxb, rows, tbl, acc, semi, semg, sems):
        c = lax.axis_index("c")
        s = lax.axis_index("s")
        grp = s if feat else c * NS + s
        if feat:
            stage_src = tbl_hbm.at[c, pl.ds(s * NTR, NTR)]
        else:
            stage_src = tbl_hbm.at[pl.ds(s * NTR, NTR)]
        pltpu.sync_copy(stage_src, tbl.at[pl.ds(s * NTR, NTR)])
        pltpu.sync_copy(zeros_hbm, acc.at[pl.ds(s * RPT, RPT)])
        pltpu.sync_copy(cmb_hbm.at[grp, pl.ds(0, SB)], idxb.at[0])
        plsc.subcore_barrier()

        def body(q, carry):
            pq = jnp.bitwise_and(q, 1)

            @pl.when(q < nsb - 1)
            def _start():
                pltpu.async_copy(cmb_hbm.at[grp, pl.ds((q + 1) * SB, SB)],
                                 idxb.at[1 - pq], semi)

            _sb_pipeline(tbl, acc, idxb.at[pq], rows, semg, sems)

            @pl.when(q < nsb - 1)
            def _wait():
                pltpu.make_async_copy(cmb_hbm.at[grp, pl.ds((q + 1) * SB, SB)],
                                      idxb.at[1 - pq], semi).wait()

            return carry

        lax.fori_loop(0, nsb, body, 0)
        plsc.subcore_barrier()
        pltpu.sync_copy(acc.at[pl.ds(s * NTR, NTR)],
                        out_hbm.at[c, pl.ds(s * NTR, NTR)])

    return k(table, cmb, zero_rows)


_ROWS = 2000


def _k0(x, w1):
    def body(x_ref, w_ref, o_ref):
        o_ref[...] = jax.lax.dot_general(
            x_ref[...], w_ref[...], (((1,), (0,)), ((), ())),
            preferred_element_type=jnp.float32,
            precision=jax.lax.Precision.HIGHEST)

    d_in, d_h = w1.shape
    return pl.pallas_call(
        body,
        grid=(N // _ROWS,),
        in_specs=[
            pl.BlockSpec((_ROWS, d_in), lambda i: (i, 0)),
            pl.BlockSpec((d_in, d_h), lambda i: (0, 0)),
        ],
        out_specs=pl.BlockSpec((_ROWS, d_h), lambda i: (i, 0)),
        out_shape=jax.ShapeDtypeStruct((N, d_h), jnp.float32),
    )(x, w1)


def _k1(p1, degt):
    def body(p_ref, deg_ref, o_ref, dinv_ref):
        dinv = lax.rsqrt(jnp.sum(deg_ref[...], axis=1, keepdims=True) + 1.0)
        dinv_ref[...] = dinv
        g = p_ref[...] * dinv
        h = p_ref.shape[1] // 2
        o_ref[0] = g[:, :h]
        o_ref[1] = g[:, h:]

    d_h = p1.shape[1]
    return pl.pallas_call(
        body,
        grid=(N // _ROWS,),
        in_specs=[
            pl.BlockSpec((_ROWS, d_h), lambda i: (i, 0)),
            pl.BlockSpec((_ROWS, NW), lambda i: (i, 0)),
        ],
        out_specs=[
            pl.BlockSpec((2, _ROWS, d_h // 2), lambda i: (0, i, 0)),
            pl.BlockSpec((_ROWS, 1), lambda i: (i, 0)),
        ],
        out_shape=[
            jax.ShapeDtypeStruct((2, N, d_h // 2), jnp.float32),
            jax.ShapeDtypeStruct((N, 1), jnp.float32),
        ],
    )(p1, degt)


def _k2(s1, g1s, dinv, b1, w2):
    def body(s_ref, g_ref, dinv_ref, b_ref, w_ref, o_ref):
        dv = dinv_ref[...]
        full = jnp.concatenate([s_ref[0] + g_ref[0], s_ref[1] + g_ref[1]],
                               axis=-1)
        h = dv * full + b_ref[...]
        h = jnp.maximum(h, 0.0)
        p = jax.lax.dot_general(h, w_ref[...], (((1,), (0,)), ((), ())),
                                preferred_element_type=jnp.float32,
                                precision=jax.lax.Precision.HIGHEST)
        o_ref[...] = p * dv

    d_h, d_o = w2.shape
    return pl.pallas_call(
        body,
        grid=(N // _ROWS,),
        in_specs=[
            pl.BlockSpec((2, _ROWS, d_h // 2), lambda i: (0, i, 0)),
            pl.BlockSpec((2, _ROWS, d_h // 2), lambda i: (0, i, 0)),
            pl.BlockSpec((_ROWS, 1), lambda i: (i, 0)),
            pl.BlockSpec((1, d_h), lambda i: (0, 0)),
            pl.BlockSpec((d_h, d_o), lambda i: (0, 0)),
        ],
        out_specs=pl.BlockSpec((_ROWS, d_o), lambda i: (i, 0)),
        out_shape=jax.ShapeDtypeStruct((N, d_o), jnp.float32),
    )(s1, g1s, dinv, b1, w2)


def _k3(s2, g2, dinv, b2):
    def body(s_ref, g_ref, dinv_ref, b_ref, o_ref):
        o_ref[...] = (dinv_ref[...] * (s_ref[0] + s_ref[1] + g_ref[...])
                      + b_ref[...])

    d_o = g2.shape[1]
    return pl.pallas_call(
        body,
        grid=(N // _ROWS,),
        in_specs=[
            pl.BlockSpec((2, _ROWS, d_o), lambda i: (0, i, 0)),
            pl.BlockSpec((_ROWS, d_o), lambda i: (i, 0)),
            pl.BlockSpec((_ROWS, 1), lambda i: (i, 0)),
            pl.BlockSpec((1, d_o), lambda i: (0, 0)),
        ],
        out_specs=pl.BlockSpec((_ROWS, d_o), lambda i: (i, 0)),
        out_shape=jax.ShapeDtypeStruct((N, d_o), jnp.float32),
    )(s2, g2, dinv, b2)


def kernel(x, edge_index, W1, b1, W2, b2):
    d_h = W1.shape[1]
    d_o = W2.shape[1]
    pad = E_PAD - E
    src = jnp.concatenate([edge_index[0], jnp.zeros((pad,), jnp.int32)])
    dst = jnp.concatenate([edge_index[1], jnp.full((pad,), N, jnp.int32)])
    dst3d = dst.reshape(NW, K, B)
    cmb = jnp.stack([src.reshape(NS, K2, B), dst.reshape(NS, K2, B)],
                    axis=2)
    cmb_edge = cmb.reshape(NW, K, 2, B)

    zeros_h = jnp.zeros((RPT, d_h // 2), jnp.float32)
    zeros_o = jnp.zeros((RPT, d_o), jnp.float32)

    deg_parts = _deg_pass(dst3d)
    degt = deg_parts.T

    p1 = _k0(x, W1)
    g1s, dinv = _k1(p1, degt)
    s1 = _seg_pass(g1s, cmb, zeros_h, d_h // 2, K2 // SB, True)
    g2 = _k2(s1, g1s, dinv, b1.reshape(1, d_h), W2)
    s2 = _seg_pass(g2, cmb_edge, zeros_o, d_o, K // SB, False)
    return _k3(s2, g2, dinv, b2.reshape(1, d_o))


# speedup vs baseline: 1.0724x; 1.0091x over previous
import dataclasses
import functools

import jax
import jax.numpy as jnp
from jax import lax
from jax.experimental import pallas as pl
from jax.experimental.pallas import tpu as pltpu
from jax.experimental.pallas import tpu_sc as plsc

NC = 2
NS = 16
NW = NC * NS
B = 128

N = 10000
E = 320000
K = 80
K2 = 2 * K
E_PAD = NW * K * B
N_PAD = 10112
RPT = N_PAD // NS


def _mesh():
    return plsc.VectorSubcoreMesh(core_axis_name="c", subcore_axis_name="s")


def _sc_params():
    cp = pltpu.CompilerParams()
    fields = pltpu.CompilerParams.__dataclass_fields__
    if "needs_layout_passes" in fields:
        cp = dataclasses.replace(cp, needs_layout_passes=False)
    if "use_tc_tiling_on_sc" in fields:
        cp = dataclasses.replace(cp, use_tc_tiling_on_sc=False)
    return cp


def _deg_pass(dst3d):
    @functools.partial(
        pl.kernel,
        out_type=jax.ShapeDtypeStruct((NW, N), jnp.float32),
        mesh=_mesh(),
        scratch_types=[
            pltpu.VMEM((K, B), jnp.int32),
            pltpu.VMEM((N_PAD,), jnp.float32),
        ],
        compiler_params=_sc_params(),
    )
    def k(dst_hbm, out_hbm, dst_v, hist):
        c = lax.axis_index("c")
        s = lax.axis_index("s")
        wid = c * NS + s
        pltpu.sync_copy(dst_hbm.at[wid], dst_v)

        def zero(i, carry):
            hist[pl.ds(i * 16, 16)] = jnp.zeros((16,), jnp.float32)
            return carry

        lax.fori_loop(0, N_PAD // 16, zero, 0)
        ones16 = jnp.ones((16,), jnp.float32)

        def body(j, carry):
            for l in range(B // 16):
                idx = dst_v[j, pl.ds(l * 16, 16)]
                plsc.addupdate_scatter(hist, [idx], ones16)
            return carry

        lax.fori_loop(0, K, body, 0)
        pltpu.sync_copy(hist.at[pl.ds(0, N)], out_hbm.at[wid])

    return k(dst3d)


W = 4
SB = 20
NTR = N // NS


def _sb_pipeline(tbl, acc, idx, rows, semg, sems):
    for b in range(W):
        pltpu.async_copy(tbl.at[idx.at[b, 0]], rows.at[b], semg.at[b])
    rounds = SB // W
    for r in range(rounds):
        for b in range(W):
            t = r * W + b
            pltpu.make_async_copy(tbl.at[idx.at[t, 0]], rows.at[b],
                                  semg.at[b]).wait()
            pltpu.async_copy(rows.at[b], acc.at[idx.at[t, 1]], sems.at[b],
                             add=True)
        for b in range(W):
            t = r * W + b
            if t + W < SB:
                pltpu.make_async_copy(rows.at[b], acc.at[idx.at[t, 1]],
                                      sems.at[b]).wait()
                pltpu.async_copy(tbl.at[idx.at[t + W, 0]], rows.at[b],
                                 semg.at[b])
    for b in range(W):
        t = (rounds - 1) * W + b
        pltpu.make_async_copy(rows.at[b], acc.at[idx.at[t, 1]],
                              sems.at[b]).wait()


def _seg_pass(table, cmb, zero_rows, d, nsb, feat):
    @functools.partial(
        pl.kernel,
        out_type=jax.ShapeDtypeStruct((NC, N, d), jnp.float32),
        mesh=_mesh(),
        scratch_types=[
            pltpu.VMEM((2, SB, 2, B), jnp.int32),
            pltpu.VMEM((W, B, d), jnp.float32),
            pltpu.VMEM_SHARED((N, d), jnp.float32),
            pltpu.VMEM_SHARED((N_PAD, d), jnp.float32),
            pltpu.SemaphoreType.DMA,
            pltpu.SemaphoreType.DMA((W,)),
            pltpu.SemaphoreType.DMA((W,)),
        ],
        compiler_params=_sc_params(),
    )
    def k(tbl_hbm, cmb_hbm, zeros_hbm, out_hbm,
          idxb, rows, tbl, acc, semi, semg, sems):
        c = lax.axis_index("c")
        s = lax.axis_index("s")
        grp = s if feat else c * NS + s
        if feat:
            stage_src = tbl_hbm.at[c, pl.ds(s * NTR, NTR)]
        else:
            stage_src = tbl_hbm.at[pl.ds(s * NTR, NTR)]
        pltpu.sync_copy(stage_src, tbl.at[pl.ds(s * NTR, NTR)])
        pltpu.sync_copy(zeros_hbm, acc.at[pl.ds(s * RPT, RPT)])
        pltpu.sync_copy(cmb_hbm.at[grp, pl.ds(0, SB)], idxb.at[0])
        plsc.subcore_barrier()

        def body(q, carry):
            pq = jnp.bitwise_and(q, 1)

            @pl.when(q < nsb - 1)
            def _start():
                pltpu.async_copy(cmb_hbm.at[grp, pl.ds((q + 1) * SB, SB)],
                                 idxb.at[1 - pq], semi)

            _sb_pipeline(tbl, acc, idxb.at[pq], rows, semg, sems)

            @pl.when(q < nsb - 1)
            def _wait():
                pltpu.make_async_copy(cmb_hbm.at[grp, pl.ds((q + 1) * SB, SB)],
                                      idxb.at[1 - pq], semi).wait()

            return carry

        lax.fori_loop(0, nsb, body, 0)
        plsc.subcore_barrier()
        pltpu.sync_copy(acc.at[pl.ds(s * NTR, NTR)],
                        out_hbm.at[c, pl.ds(s * NTR, NTR)])

    return k(table, cmb, zero_rows)


_ROWS = 2000


def _k0(x, w1):
    def body(x_ref, w_ref, o_ref):
        o_ref[...] = jax.lax.dot_general(
            x_ref[...], w_ref[...], (((1,), (0,)), ((), ())),
            preferred_element_type=jnp.float32,
            precision=jax.lax.Precision.HIGHEST)

    d_in, d_h = w1.shape
    return pl.pallas_call(
        body,
        grid=(N // _ROWS,),
        in_specs=[
            pl.BlockSpec((_ROWS, d_in), lambda i: (i, 0)),
            pl.BlockSpec((d_in, d_h), lambda i: (0, 0)),
        ],
        out_specs=pl.BlockSpec((_ROWS, d_h), lambda i: (i, 0)),
        out_shape=jax.ShapeDtypeStruct((N, d_h), jnp.float32),
    )(x, w1)


def _k1(p1, degt):
    def body(p_ref, deg_ref, o_ref, dinv_ref):
        dinv = lax.rsqrt(jnp.sum(deg_ref[...], axis=1, keepdims=True) + 1.0)
        dinv_ref[...] = dinv
        g = p_ref[...] * dinv
        h = p_ref.shape[1] // 2
        o_ref[0] = g[:, :h]
        o_ref[1] = g[:, h:]

    d_h = p1.shape[1]
    return pl.pallas_call(
        body,
        grid=(N // _ROWS,),
        in_specs=[
            pl.BlockSpec((_ROWS, d_h), lambda i: (i, 0)),
            pl.BlockSpec((_ROWS, NW), lambda i: (i, 0)),
        ],
        out_specs=[
            pl.BlockSpec((2, _ROWS, d_h // 2), lambda i: (0, i, 0)),
            pl.BlockSpec((_ROWS, 1), lambda i: (i, 0)),
        ],
        out_shape=[
            jax.ShapeDtypeStruct((2, N, d_h // 2), jnp.float32),
            jax.ShapeDtypeStruct((N, 1), jnp.float32),
        ],
    )(p1, degt)


def _k2(s1, g1s, dinv, b1, w2):
    def body(s_ref, g_ref, dinv_ref, b_ref, w_ref, o_ref):
        dv = dinv_ref[...]
        full = jnp.concatenate([s_ref[0] + g_ref[0], s_ref[1] + g_ref[1]],
                               axis=-1)
        h = dv * full + b_ref[...]
        h = jnp.maximum(h, 0.0)
        p = jax.lax.dot_general(h, w_ref[...], (((1,), (0,)), ((), ())),
                                preferred_element_type=jnp.float32,
                                precision=jax.lax.Precision.HIGHEST)
        o_ref[...] = p * dv

    d_h, d_o = w2.shape
    return pl.pallas_call(
        body,
        grid=(N // _ROWS,),
        in_specs=[
            pl.BlockSpec((2, _ROWS, d_h // 2), lambda i: (0, i, 0)),
            pl.BlockSpec((2, _ROWS, d_h // 2), lambda i: (0, i, 0)),
            pl.BlockSpec((_ROWS, 1), lambda i: (i, 0)),
            pl.BlockSpec((1, d_h), lambda i: (0, 0)),
            pl.BlockSpec((d_h, d_o), lambda i: (0, 0)),
        ],
        out_specs=pl.BlockSpec((_ROWS, d_o), lambda i: (i, 0)),
        out_shape=jax.ShapeDtypeStruct((N, d_o), jnp.float32),
    )(s1, g1s, dinv, b1, w2)


def _k3(s2, g2, dinv, b2):
    def body(s_ref, g_ref, dinv_ref, b_ref, o_ref):
        o_ref[...] = (dinv_ref[...] * (s_ref[0] + s_ref[1] + g_ref[...])
                      + b_ref[...])

    d_o = g2.shape[1]
    return pl.pallas_call(
        body,
        grid=(N // _ROWS,),
        in_specs=[
            pl.BlockSpec((2, _ROWS, d_o), lambda i: (0, i, 0)),
            pl.BlockSpec((_ROWS, d_o), lambda i: (i, 0)),
            pl.BlockSpec((_ROWS, 1), lambda i: (i, 0)),
            pl.BlockSpec((1, d_o), lambda i: (0, 0)),
        ],
        out_specs=pl.BlockSpec((_ROWS, d_o), lambda i: (i, 0)),
        out_shape=jax.ShapeDtypeStruct((N, d_o), jnp.float32),
    )(s2, g2, dinv, b2)


def kernel(x, edge_index, W1, b1, W2, b2):
    d_h = W1.shape[1]
    d_o = W2.shape[1]
    pad = E_PAD - E
    src = jnp.concatenate([edge_index[0], jnp.zeros((pad,), jnp.int32)])
    dst = jnp.concatenate([edge_index[1], jnp.full((pad,), N, jnp.int32)])
    dst3d = dst.reshape(NW, K, B)
    cmb = jnp.stack([src.reshape(NS, K2, B), dst.reshape(NS, K2, B)],
                    axis=2)
    cmb_edge = cmb.reshape(NW, K, 2, B)

    zeros_h = jnp.zeros((RPT, d_h // 2), jnp.float32)
    zeros_o = jnp.zeros((RPT, d_o), jnp.float32)

    deg_parts = _deg_pass(dst3d)
    degt = deg_parts.T

    p1 = _k0(x, W1)
    g1s, dinv = _k1(p1, degt)
    s1 = _seg_pass(g1s, cmb, zeros_h, d_h // 2, K2 // SB, True)
    g2 = _k2(s1, g1s, dinv, b1.reshape(1, d_h), W2)
    s2 = _seg_pass(g2, cmb_edge, zeros_o, d_o, K // SB, False)
    return _k3(s2, g2, dinv, b2.reshape(1, d_o))
